# baseline (speedup 1.0000x reference)
import dataclasses
import functools

import jax
import jax.numpy as jnp
from jax import lax
from jax.experimental import pallas as pl
from jax.experimental.pallas import tpu as pltpu
from jax.experimental.pallas import tpu_sc as plsc

_BN = 512


def _sc_gather(w_packed, H):
    FOUT, FIN = H.shape
    KW = w_packed.shape[0]
    mesh = plsc.VectorSubcoreMesh(core_axis_name="c", subcore_axis_name="s")
    cp = pltpu.CompilerParams()
    if "needs_layout_passes" in pltpu.CompilerParams.__dataclass_fields__:
        cp = dataclasses.replace(cp, needs_layout_passes=False)

    @functools.partial(
        pl.kernel,
        out_type=jax.ShapeDtypeStruct((FOUT, FIN // 2), jnp.int32),
        mesh=mesh,
        scratch_types=[pltpu.VMEM((KW,), jnp.int32)],
        compiler_params=cp,
    )
    def k(w_hbm, h_hbm, o_hbm, w_vmem):
        pltpu.sync_copy(w_hbm, w_vmem)

        def body(h_vmem, o_vmem):
            for r in range(2):
                @plsc.parallel_loop(0, FIN // 32, unroll=8)
                def _(g, r=r):
                    c = g * 32
                    idx_a = h_vmem[r, pl.ds(c, 16)]
                    idx_b = h_vmem[r, pl.ds(c + 16, 16)]
                    g_a = plsc.load_gather(w_vmem, [lax.shift_right_logical(idx_a, 1)])
                    g_b = plsc.load_gather(w_vmem, [lax.shift_right_logical(idx_b, 1)])
                    sh_a = lax.shift_left(jnp.bitwise_and(idx_a, 1), 4)
                    sh_b = lax.shift_left(jnp.bitwise_and(idx_b, 1), 4)
                    bits_a = jnp.bitwise_and(lax.shift_right_logical(g_a, sh_a), 0xFFFF)
                    bits_b = lax.shift_left(lax.shift_right_logical(g_b, sh_b), 16)
                    o_vmem[r, pl.ds(g * 16, 16)] = jnp.bitwise_or(bits_a, bits_b)

        pltpu.emit_pipeline(
            body,
            grid=(FOUT // 2,),
            in_specs=[pl.BlockSpec((2, FIN), lambda i: (i, 0))],
            out_specs=[pl.BlockSpec((2, FIN // 2), lambda i: (i, 0))],
            core_axis_name=("c", "s"),
            dimension_semantics=(pltpu.PARALLEL,),
        )(h_hbm, o_hbm)

    return k(w_packed, H)


def _matmul_kernel(al_ref, ah_ref, w_ref, o_ref):
    wi = w_ref[...]
    w_lo = lax.bitcast_convert_type(
        lax.shift_left(wi, 16), jnp.float32).astype(jnp.bfloat16)
    w_hi = lax.bitcast_convert_type(wi, jnp.float32).astype(jnp.bfloat16)
    dn = (((1,), (1,)), ((), ()))
    o_ref[...] = (
        lax.dot_general(al_ref[...], w_lo, dn, preferred_element_type=jnp.float32)
        + lax.dot_general(ah_ref[...], w_hi, dn, preferred_element_type=jnp.float32))


_NCHUNK = 4


def kernel(a, W, H):
    B, FIN = a.shape
    FOUT = H.shape[0]
    wb = W.astype(jnp.bfloat16)
    w_packed = lax.bitcast_convert_type(wb.reshape(-1, 2), jnp.int32)
    a_sp = a.reshape(B, FIN // 32, 2, 16).astype(jnp.bfloat16)
    a_lo = a_sp[:, :, 0, :].reshape(B, FIN // 2)
    a_hi = a_sp[:, :, 1, :].reshape(B, FIN // 2)
    rows = FOUT // _NCHUNK
    zs = []
    for c in range(_NCHUNK):
        wmat_i32 = _sc_gather(w_packed, lax.slice_in_dim(H, c * rows, (c + 1) * rows))
        zs.append(pl.pallas_call(
            _matmul_kernel,
            grid=(rows // _BN,),
            in_specs=[pl.BlockSpec((B, FIN // 2), lambda j: (0, 0)),
                      pl.BlockSpec((B, FIN // 2), lambda j: (0, 0)),
                      pl.BlockSpec((_BN, FIN // 2), lambda j: (j, 0))],
            out_specs=pl.BlockSpec((B, _BN), lambda j: (0, j)),
            out_shape=jax.ShapeDtypeStruct((B, rows), jnp.float32),
        )(a_lo, a_hi, wmat_i32))
    return jnp.concatenate(zs, axis=1)


# speedup vs baseline: 1.5779x; 1.5779x over previous
import dataclasses
import functools

import jax
import jax.numpy as jnp
from jax import lax
from jax.experimental import pallas as pl
from jax.experimental.pallas import tpu as pltpu
from jax.experimental.pallas import tpu_sc as plsc

_BN = 512


def _sc_gather(w_packed, H):
    FOUT, FIN = H.shape
    KW = w_packed.shape[0]
    mesh = plsc.VectorSubcoreMesh(core_axis_name="c", subcore_axis_name="s")
    cp = pltpu.CompilerParams()
    if "needs_layout_passes" in pltpu.CompilerParams.__dataclass_fields__:
        cp = dataclasses.replace(cp, needs_layout_passes=False)

    @functools.partial(
        pl.kernel,
        out_type=jax.ShapeDtypeStruct((FOUT, FIN // 2), jnp.int32),
        mesh=mesh,
        scratch_types=[pltpu.VMEM((KW,), jnp.int32)],
        compiler_params=cp,
    )
    def k(w_hbm, h_hbm, o_hbm, w_vmem):
        pltpu.sync_copy(w_hbm, w_vmem)

        def body(h_vmem, o_vmem):
            for r in range(2):
                @plsc.parallel_loop(0, FIN // 32, unroll=8)
                def _(g, r=r):
                    c = g * 32
                    idx_a = h_vmem[r, pl.ds(c, 16)]
                    idx_b = h_vmem[r, pl.ds(c + 16, 16)]
                    g_a = plsc.load_gather(w_vmem, [lax.shift_right_logical(idx_a, 1)])
                    g_b = plsc.load_gather(w_vmem, [lax.shift_right_logical(idx_b, 1)])
                    sh_a = lax.shift_left(jnp.bitwise_and(idx_a, 1), 4)
                    sh_b = lax.shift_left(jnp.bitwise_and(idx_b, 1), 4)
                    bits_a = jnp.bitwise_and(lax.shift_right_logical(g_a, sh_a), 0xFFFF)
                    bits_b = lax.shift_left(lax.shift_right_logical(g_b, sh_b), 16)
                    o_vmem[r, pl.ds(g * 16, 16)] = jnp.bitwise_or(bits_a, bits_b)

        pltpu.emit_pipeline(
            body,
            grid=(FOUT // 2,),
            in_specs=[pl.BlockSpec((2, FIN), lambda i: (i, 0))],
            out_specs=[pl.BlockSpec((2, FIN // 2), lambda i: (i, 0))],
            core_axis_name=("c", "s"),
            dimension_semantics=(pltpu.PARALLEL,),
        )(h_hbm, o_hbm)

    return k(w_packed, H)


def _matmul_kernel(al_ref, ah_ref, w_ref, o_ref):
    wi = w_ref[...]
    w_lo = lax.bitcast_convert_type(
        lax.shift_left(wi, 16), jnp.float32).astype(jnp.bfloat16)
    w_hi = lax.bitcast_convert_type(wi, jnp.float32).astype(jnp.bfloat16)
    dn = (((1,), (1,)), ((), ()))
    o_ref[...] = (
        lax.dot_general(al_ref[...], w_lo, dn, preferred_element_type=jnp.float32)
        + lax.dot_general(ah_ref[...], w_hi, dn, preferred_element_type=jnp.float32))


def kernel(a, W, H):
    B, FIN = a.shape
    FOUT = H.shape[0]
    wb = W.astype(jnp.bfloat16)
    w_packed = lax.bitcast_convert_type(wb.reshape(-1, 2), jnp.int32)
    wmat_i32 = _sc_gather(w_packed, H)
    a_sp = a.reshape(B, FIN // 32, 2, 16).astype(jnp.bfloat16)
    a_lo = a_sp[:, :, 0, :].reshape(B, FIN // 2)
    a_hi = a_sp[:, :, 1, :].reshape(B, FIN // 2)
    z = pl.pallas_call(
        _matmul_kernel,
        grid=(FOUT // _BN,),
        in_specs=[pl.BlockSpec((B, FIN // 2), lambda j: (0, 0)),
                  pl.BlockSpec((B, FIN // 2), lambda j: (0, 0)),
                  pl.BlockSpec((_BN, FIN // 2), lambda j: (j, 0))],
        out_specs=pl.BlockSpec((B, _BN), lambda j: (0, j)),
        out_shape=jax.ShapeDtypeStruct((B, FOUT), jnp.float32),
    )(a_lo, a_hi, wmat_i32)
    return z


# speedup vs baseline: 1.5836x; 1.0036x over previous
import dataclasses
import functools

import jax
import jax.numpy as jnp
from jax import lax
from jax.experimental import pallas as pl
from jax.experimental.pallas import tpu as pltpu
from jax.experimental.pallas import tpu_sc as plsc

_BR = 256


def _sc_gather(w_packed, H):
    FOUT, FIN = H.shape
    KW = w_packed.shape[0]
    mesh = plsc.VectorSubcoreMesh(core_axis_name="c", subcore_axis_name="s")
    cp = pltpu.CompilerParams()
    if "needs_layout_passes" in pltpu.CompilerParams.__dataclass_fields__:
        cp = dataclasses.replace(cp, needs_layout_passes=False)

    @functools.partial(
        pl.kernel,
        out_type=jax.ShapeDtypeStruct((FOUT // 2, FIN), jnp.int32),
        mesh=mesh,
        scratch_types=[pltpu.VMEM((KW,), jnp.int32)],
        compiler_params=cp,
    )
    def k(w_hbm, h_hbm, o_hbm, w_vmem):
        pltpu.sync_copy(w_hbm, w_vmem)

        def body(h_vmem, o_vmem):
            @plsc.parallel_loop(0, FIN // 16, unroll=8)
            def _(g):
                c = g * 16
                idx_a = h_vmem[0, pl.ds(c, 16)]
                idx_b = h_vmem[1, pl.ds(c, 16)]
                g_a = plsc.load_gather(w_vmem, [lax.shift_right_logical(idx_a, 1)])
                g_b = plsc.load_gather(w_vmem, [lax.shift_right_logical(idx_b, 1)])
                sh_a = lax.shift_left(jnp.bitwise_and(idx_a, 1), 4)
                sh_b = lax.shift_left(jnp.bitwise_and(idx_b, 1), 4)
                bits_a = jnp.bitwise_and(lax.shift_right_logical(g_a, sh_a), 0xFFFF)
                bits_b = lax.shift_left(lax.shift_right_logical(g_b, sh_b), 16)
                o_vmem[0, pl.ds(c, 16)] = jnp.bitwise_or(bits_a, bits_b)

        pltpu.emit_pipeline(
            body,
            grid=(FOUT // 2,),
            in_specs=[pl.BlockSpec((2, FIN), lambda i: (i, 0))],
            out_specs=[pl.BlockSpec((1, FIN), lambda i: (i, 0))],
            core_axis_name=("c", "s"),
            dimension_semantics=(pltpu.PARALLEL,),
        )(h_hbm, o_hbm)

    return k(w_packed, H)


def _matmul_kernel(a_ref, x_ref, o_ref):
    wb = pltpu.bitcast(x_ref[...], jnp.bfloat16)
    o_ref[...] = lax.dot_general(
        a_ref[...], wb, (((1,), (1,)), ((), ())),
        preferred_element_type=jnp.float32)


def kernel(a, W, H):
    B, FIN = a.shape
    FOUT = H.shape[0]
    wb16 = W.astype(jnp.bfloat16)
    w_packed = lax.bitcast_convert_type(wb16.reshape(-1, 2), jnp.int32)
    x_i32 = _sc_gather(w_packed, H)
    ab = a.astype(jnp.bfloat16)
    z = pl.pallas_call(
        _matmul_kernel,
        grid=(FOUT // (2 * _BR),),
        in_specs=[pl.BlockSpec((B, FIN), lambda j: (0, 0)),
                  pl.BlockSpec((_BR, FIN), lambda j: (j, 0))],
        out_specs=pl.BlockSpec((B, 2 * _BR), lambda j: (0, j)),
        out_shape=jax.ShapeDtypeStruct((B, FOUT), jnp.float32),
    )(ab, x_i32)
    return z


# speedup vs baseline: 1.8711x; 1.1815x over previous
import dataclasses
import functools

import jax
import jax.numpy as jnp
from jax import lax
from jax.experimental import pallas as pl
from jax.experimental.pallas import tpu as pltpu
from jax.experimental.pallas import tpu_sc as plsc

_BR = 512


def _pack_kernel(w_ref, o_ref):
    kw = o_ref.shape[0]
    lo = lax.bitcast_convert_type(w_ref[pl.ds(0, kw)], jnp.uint32)
    hi = lax.bitcast_convert_type(w_ref[pl.ds(kw, kw)], jnp.uint32)

    def rnd16(u):
        odd = jnp.bitwise_and(lax.shift_right_logical(u, jnp.uint32(16)),
                              jnp.uint32(1))
        return lax.shift_right_logical(u + jnp.uint32(0x7FFF) + odd,
                                       jnp.uint32(16))

    o_ref[...] = lax.bitcast_convert_type(
        jnp.bitwise_or(rnd16(lo),
                       lax.shift_left(rnd16(hi), jnp.uint32(16))), jnp.int32)


def _sc_gather(w_packed, H):
    FOUT, FIN = H.shape
    KW = w_packed.shape[0]
    mesh = plsc.VectorSubcoreMesh(core_axis_name="c", subcore_axis_name="s")
    cp = pltpu.CompilerParams()
    if "needs_layout_passes" in pltpu.CompilerParams.__dataclass_fields__:
        cp = dataclasses.replace(cp, needs_layout_passes=False)

    @functools.partial(
        pl.kernel,
        out_type=jax.ShapeDtypeStruct((FOUT // 2, FIN), jnp.int32),
        mesh=mesh,
        scratch_types=[pltpu.VMEM((KW,), jnp.int32)],
        compiler_params=cp,
    )
    def k(w_hbm, h_hbm, o_hbm, w_vmem):
        pltpu.sync_copy(w_hbm, w_vmem)

        def body(h_vmem, o_vmem):
            @plsc.parallel_loop(0, FIN // 16, unroll=8)
            def _(g):
                c = g * 16
                idx_a = h_vmem[0, pl.ds(c, 16)]
                idx_b = h_vmem[1, pl.ds(c, 16)]
                g_a = plsc.load_gather(w_vmem, [jnp.bitwise_and(idx_a, KW - 1)])
                g_b = plsc.load_gather(w_vmem, [jnp.bitwise_and(idx_b, KW - 1)])
                sh_a = jnp.bitwise_and(lax.shift_right_logical(idx_a, 12), 16)
                sh_b = jnp.bitwise_and(lax.shift_right_logical(idx_b, 12), 16)
                bits_a = jnp.bitwise_and(lax.shift_right_logical(g_a, sh_a), 0xFFFF)
                bits_b = lax.shift_left(lax.shift_right_logical(g_b, sh_b), 16)
                o_vmem[0, pl.ds(c, 16)] = jnp.bitwise_or(bits_a, bits_b)

        pltpu.emit_pipeline(
            body,
            grid=(FOUT // 2,),
            in_specs=[pl.BlockSpec((2, FIN), lambda i: (i, 0))],
            out_specs=[pl.BlockSpec((1, FIN), lambda i: (i, 0))],
            core_axis_name=("c", "s"),
            dimension_semantics=(pltpu.PARALLEL,),
        )(h_hbm, o_hbm)

    return k(w_packed, H)


def _matmul_kernel(a_ref, x_ref, o_ref):
    wb = pltpu.bitcast(x_ref[...], jnp.bfloat16)
    o_ref[...] = lax.dot_general(
        a_ref[...], wb, (((1,), (1,)), ((), ())),
        preferred_element_type=jnp.float32)


def kernel(a, W, H):
    B, FIN = a.shape
    FOUT = H.shape[0]
    KW = W.shape[0] // 2
    w_packed = pl.pallas_call(
        _pack_kernel,
        out_shape=jax.ShapeDtypeStruct((KW,), jnp.int32),
    )(W)
    x_i32 = _sc_gather(w_packed, H)
    ab = a.astype(jnp.bfloat16)
    z = pl.pallas_call(
        _matmul_kernel,
        grid=(FOUT // (2 * _BR),),
        in_specs=[pl.BlockSpec((B, FIN), lambda j: (0, 0)),
                  pl.BlockSpec((_BR, FIN), lambda j: (j, 0))],
        out_specs=pl.BlockSpec((B, 2 * _BR), lambda j: (0, j)),
        out_shape=jax.ShapeDtypeStruct((B, FOUT), jnp.float32),
    )(ab, x_i32)
    return z
